# baseline (speedup 1.0000x reference)
import jax
import jax.numpy as jnp
from jax.experimental import pallas as pl
from jax.experimental.pallas import tpu as pltpu

_TAU = 0.5
_EPS = 1e-10
_LOG2E = 1.4426950408889634
_LN2 = 0.6931471805599453


def _ccl_body(x_ref, out_ref, colsum_ref, rsum_ref, diag_ref, acc_ref,
              eye_ref, cache_ref):
    phase = pl.program_id(0)
    step = pl.program_id(1)
    nsteps = pl.num_programs(1)
    b = eye_ref.shape[0]
    n = cache_ref.shape[1]
    c = _LOG2E / _TAU

    @pl.when(phase == 0)
    def _sums():
        @pl.when(step == 0)
        def _init():
            colsum_ref[...] = jnp.zeros_like(colsum_ref)
            acc_ref[...] = jnp.zeros_like(acc_ref)
            r = jax.lax.broadcasted_iota(jnp.int32, (b, b), 0)
            cc = jax.lax.broadcasted_iota(jnp.int32, (b, b), 1)
            eye_ref[...] = jnp.where(r == cc, 1.0, 0.0)

        w = n // 32
        rs = jnp.zeros((b, 1), jnp.float32)
        for k in range(32):
            ssl = jnp.exp2(x_ref[:, k * w:(k + 1) * w] * c)
            colsum_ref[0:1, k * w:(k + 1) * w] += ssl.sum(axis=0, keepdims=True)
            rs = rs + ssl.sum(axis=1, keepdims=True)
            cache_ref[pl.ds(step * b, b), k * w:(k + 1) * w] = ssl.astype(
                jnp.bfloat16)

        rinv = 1.0 / (rs + _EPS)
        rsum_ref[0:1, pl.ds(step * b, b)] = jnp.swapaxes(rs, 0, 1)
        dtile = jnp.exp2(x_ref[:, pl.ds(step * b, b)] * c)
        diag_ref[0:1, pl.ds(step * b, b)] = (
            dtile * eye_ref[...]).sum(axis=0, keepdims=True)

        def zrow(k):
            sq = cache_ref[pl.ds(step * b, b), k * w:(k + 1) * w].astype(
                jnp.float32)
            return 1.0 - sq * rinv

        tsum = jnp.zeros((b, 1), jnp.float32)
        for g in range(8):
            k = 4 * g
            term = jnp.log2((zrow(k) * zrow(k + 1))
                            * (zrow(k + 2) * zrow(k + 3)))
            tsum = tsum + term.sum(axis=1, keepdims=True)
        acc_ref[...] += tsum.sum(axis=0, keepdims=True)

    @pl.when(phase == 1)
    def _loss():
        @pl.when(step == 0)
        def _recip():
            colsum_ref[...] = 1.0 / (colsum_ref[...] + _EPS)

        rinvT = 1.0 / (rsum_ref[0:1, pl.ds(step * b, b)] + _EPS)
        w = n // 32

        def zcol(k):
            sq = cache_ref[pl.ds(step * b, b), k * w:(k + 1) * w].astype(
                jnp.float32)
            cq = colsum_ref[0:1, k * w:(k + 1) * w]
            return 1.0 - sq * cq

        tsum = jnp.zeros((b, 1), jnp.float32)
        for g in range(8):
            k = 4 * g
            term = jnp.log2((zcol(k) * zcol(k + 1))
                            * (zcol(k + 2) * zcol(k + 3)))
            tsum = tsum + term.sum(axis=1, keepdims=True)

        dT = diag_ref[0:1, pl.ds(step * b, b)]
        cinvT = colsum_ref[0:1, pl.ds(step * b, b)]
        dcorr = jnp.log2((1.0 - dT * rinvT) * (1.0 - dT * cinvT))

        acc_ref[...] += (tsum.sum(axis=0, keepdims=True)
                         - dcorr.sum(axis=1, keepdims=True))

        @pl.when(step == nsteps - 1)
        def _finish():
            out_ref[...] = acc_ref[...] * (-_LN2 / n)


def kernel(scores):
    n = scores.shape[0]
    block = 512
    nsteps = n // block
    grid = (2, nsteps)
    out = pl.pallas_call(
        _ccl_body,
        grid=grid,
        in_specs=[pl.BlockSpec(
            (block, n),
            lambda p, i: (jnp.where(p == 0, i, nsteps - 1), 0))],
        out_specs=pl.BlockSpec((1, 1), lambda p, i: (0, 0)),
        out_shape=jax.ShapeDtypeStruct((1, 1), jnp.float32),
        scratch_shapes=[
            pltpu.VMEM((1, n), jnp.float32),
            pltpu.VMEM((1, n), jnp.float32),
            pltpu.VMEM((1, n), jnp.float32),
            pltpu.VMEM((1, 1), jnp.float32),
            pltpu.VMEM((block, block), jnp.float32),
            pltpu.VMEM((n, n), jnp.bfloat16),
        ],
        compiler_params=pltpu.CompilerParams(
            dimension_semantics=("arbitrary", "arbitrary"),
        ),
    )(scores)
    return out[0, 0]


# speedup vs baseline: 1.1699x; 1.1699x over previous
import jax
import jax.numpy as jnp
from jax.experimental import pallas as pl
from jax.experimental.pallas import tpu as pltpu

_TAU = 0.5
_EPS = 1e-10
_LOG2E = 1.4426950408889634
_LN2 = 0.6931471805599453


def _ccl_body(x_ref, out_ref, colsum_ref, rsum_ref, diag_ref, acc_ref,
              eye_ref, cache_ref):
    phase = pl.program_id(0)
    step = pl.program_id(1)
    nsteps = pl.num_programs(1)
    b = eye_ref.shape[0]
    n = cache_ref.shape[1]
    c = _LOG2E / _TAU

    @pl.when(phase == 0)
    def _sums():
        @pl.when(step == 0)
        def _init():
            colsum_ref[...] = jnp.zeros_like(colsum_ref)
            acc_ref[...] = jnp.zeros_like(acc_ref)
            r = jax.lax.broadcasted_iota(jnp.int32, (b, b), 0)
            cc = jax.lax.broadcasted_iota(jnp.int32, (b, b), 1)
            eye_ref[...] = jnp.where(r == cc, 1.0, 0.0)

        s = jnp.exp2(x_ref[...] * c)
        colsum_ref[...] += s.sum(axis=0, keepdims=True)
        rsum_ref[0:1, pl.ds(step * b, b)] = jnp.swapaxes(
            s.sum(axis=1, keepdims=True), 0, 1)
        dtile = jnp.exp2(x_ref[:, pl.ds(step * b, b)] * c)
        diag_ref[0:1, pl.ds(step * b, b)] = (
            dtile * eye_ref[...]).sum(axis=0, keepdims=True)
        cache_ref[pl.ds(step * b, b), :] = s.astype(jnp.bfloat16)

    @pl.when(phase == 1)
    def _loss():
        @pl.when(step == 0)
        def _recip():
            colsum_ref[...] = 1.0 / (colsum_ref[...] + _EPS)

        rinvT = 1.0 / (rsum_ref[0:1, pl.ds(step * b, b)] + _EPS)
        rinv = jnp.swapaxes(rinvT, 0, 1)
        w = n // 32

        def zslice(k):
            sq = cache_ref[pl.ds(step * b, b), k * w:(k + 1) * w].astype(
                jnp.float32)
            cq = colsum_ref[0:1, k * w:(k + 1) * w]
            return (1.0 - sq * rinv) * (1.0 - sq * cq)

        tsum = jnp.zeros((b, 1), jnp.float32)
        for g in range(4):
            k = 8 * g
            p0 = (zslice(k) * zslice(k + 1)) * (zslice(k + 2) * zslice(k + 3))
            p1 = (zslice(k + 4) * zslice(k + 5)) * (zslice(k + 6) * zslice(k + 7))
            term = jnp.log2(p0 * p1)
            tsum = tsum + term.sum(axis=1, keepdims=True)

        dT = diag_ref[0:1, pl.ds(step * b, b)]
        cinvT = colsum_ref[0:1, pl.ds(step * b, b)]
        dcorr = jnp.log2((1.0 - dT * rinvT) * (1.0 - dT * cinvT))

        acc_ref[...] += (tsum.sum(axis=0, keepdims=True)
                         - dcorr.sum(axis=1, keepdims=True))

        @pl.when(step == nsteps - 1)
        def _finish():
            out_ref[...] = acc_ref[...] * (-_LN2 / n)


def kernel(scores):
    n = scores.shape[0]
    block = 512
    nsteps = n // block
    grid = (2, nsteps)
    out = pl.pallas_call(
        _ccl_body,
        grid=grid,
        in_specs=[pl.BlockSpec(
            (block, n),
            lambda p, i: (jnp.where(p == 0, i, nsteps - 1), 0))],
        out_specs=pl.BlockSpec((1, 1), lambda p, i: (0, 0)),
        out_shape=jax.ShapeDtypeStruct((1, 1), jnp.float32),
        scratch_shapes=[
            pltpu.VMEM((1, n), jnp.float32),
            pltpu.VMEM((1, n), jnp.float32),
            pltpu.VMEM((1, n), jnp.float32),
            pltpu.VMEM((1, 1), jnp.float32),
            pltpu.VMEM((block, block), jnp.float32),
            pltpu.VMEM((n, n), jnp.bfloat16),
        ],
        compiler_params=pltpu.CompilerParams(
            dimension_semantics=("arbitrary", "arbitrary"),
        ),
    )(scores)
    return out[0, 0]
